# baseline (speedup 1.0000x reference)
import functools

import jax
import jax.numpy as jnp
from jax import lax
from jax.experimental import pallas as pl
from jax.experimental.pallas import tpu as pltpu
from jax.experimental.pallas import tpu_sc as plsc

N = 10000
NP = 10240
D = 256
DH = 128
E = 160000
EP = 163840
P = 65536
CH = 128
NC = 2
NS = 16
RT = NP // NS
ECH = EP // CH
DCH = ECH // NC // NS
MCH = 80
MECH = EP // MCH
MET = MECH // NS
DEC_CH = 64
PCH = P // DEC_CH
PT = PCH // NS
BR = 1024
BP = 2048

_MESH = plsc.VectorSubcoreMesh(core_axis_name="c", subcore_axis_name="s",
                               num_cores=NC, num_subcores=NS)


@functools.partial(
    pl.kernel,
    out_type=jax.ShapeDtypeStruct((NC * NP,), jnp.float32),
    mesh=_MESH,
    scratch_types=[
        pltpu.VMEM((DCH, CH), jnp.int32),
        pltpu.VMEM((CH,), jnp.float32),
        pltpu.VMEM((RT,), jnp.float32),
        pltpu.VMEM_SHARED((NP,), jnp.float32),
        pltpu.SemaphoreType.DMA,
    ],
)
def _deg_kernel(dstc_hbm, out_hbm, idx_v, ones_v, zeros_v, hist_sh, sd):
    c = lax.axis_index("c")
    s = lax.axis_index("s")
    for j in range(CH // 16):
        ones_v[pl.ds(j * 16, 16)] = jnp.full((16,), 1.0, jnp.float32)
    for j in range(RT // 16):
        zeros_v[pl.ds(j * 16, 16)] = jnp.zeros((16,), jnp.float32)
    pltpu.sync_copy(zeros_v, hist_sh.at[pl.ds(s * RT, RT)])
    bc = pl.multiple_of((c * NS + s) * DCH, DCH)
    pltpu.sync_copy(dstc_hbm.at[pl.ds(bc, DCH)], idx_v)
    plsc.subcore_barrier()
    for b in range(DCH // 8):
        descs = [
            pltpu.async_copy(ones_v, hist_sh.at[idx_v.at[b * 8 + q]], sd,
                             add=True)
            for q in range(8)
        ]
        for d in descs:
            d.wait()
    plsc.subcore_barrier()
    out_off = pl.multiple_of(c * NP + s * RT, RT)
    pltpu.sync_copy(hist_sh.at[pl.ds(s * RT, RT)], out_hbm.at[pl.ds(out_off, RT)])


_MP_DEPTH = 4
_NPAIR = MET // 2


@functools.partial(
    pl.kernel,
    out_type=jax.ShapeDtypeStruct((NC * NP, DH), jnp.float32),
    mesh=_MESH,
    scratch_types=(
        [pltpu.VMEM((4, 2, 2, MCH), jnp.int32),
         pltpu.VMEM((_MP_DEPTH, MCH, DH), jnp.float32),
         pltpu.VMEM_SHARED((NP, DH), jnp.float32)]
        + [pltpu.SemaphoreType.DMA] * 12
    ),
)
def _mp_kernel(h_hbm, sd_hbm, acc_hbm, sd_v, rows_v, acc_sh, *sems):
    sg = sems[0:4]
    ss = sems[4:8]
    si = sems[8:12]
    c = lax.axis_index("c")
    s = lax.axis_index("s")
    r0 = s * RT
    h_off = pl.multiple_of(c * NP + r0, RT)
    pltpu.sync_copy(h_hbm.at[pl.ds(h_off, RT)], acc_sh.at[pl.ds(r0, RT)])
    plsc.subcore_barrier()
    base = c * MECH + s * MET

    def g_issue(beta, a, b):
        for q, o in ((a, 0), (b, 1)):
            pltpu.async_copy(h_hbm.at[sd_v.at[beta].at[o].at[0]],
                             rows_v.at[q], sg[q])

    def g_wait(beta, a, b):
        for q, o in ((a, 0), (b, 1)):
            pltpu.make_async_copy(h_hbm.at[sd_v.at[beta].at[o].at[0]],
                                  rows_v.at[q], sg[q]).wait()

    def s_issue(beta, a, b):
        for q, o in ((a, 0), (b, 1)):
            pltpu.async_copy(rows_v.at[q],
                             acc_sh.at[sd_v.at[beta].at[o].at[1]], ss[q],
                             add=True)

    def s_wait(beta, a, b):
        for q, o in ((a, 0), (b, 1)):
            pltpu.make_async_copy(rows_v.at[q],
                                  acc_sh.at[sd_v.at[beta].at[o].at[1]],
                                  ss[q]).wait()

    def i_fetch(j, beta):
        jj = pl.multiple_of(2 * j, 2)
        for o in (0, 1):
            pltpu.async_copy(sd_hbm.at[base + jj + o], sd_v.at[beta].at[o],
                             si[beta])

    def i_wait(j, beta):
        jj = pl.multiple_of(2 * j, 2)
        for o in (0, 1):
            pltpu.make_async_copy(sd_hbm.at[base + jj + o],
                                  sd_v.at[beta].at[o], si[beta]).wait()

    def i_sync(j, beta):
        jj = pl.multiple_of(2 * j, 2)
        for o in (0, 1):
            pltpu.sync_copy(sd_hbm.at[base + jj + o], sd_v.at[beta].at[o])

    def subiter(j, a, b, oa, ob, beta, w_ss=True, fetch=True, w_si=True):
        g_wait(beta, a, b)
        s_issue(beta, a, b)
        if w_ss:
            s_wait((beta + 3) % 4, oa, ob)
        if fetch:
            i_fetch(j + 3, (beta + 3) % 4)
        if w_si:
            i_wait(j + 1, (beta + 1) % 4)
        g_issue((beta + 1) % 4, oa, ob)

    i_sync(0, 0)
    i_sync(1, 1)
    i_sync(2, 2)
    g_issue(0, 0, 1)
    subiter(0, 0, 1, 2, 3, 0, w_ss=False, w_si=False)
    subiter(1, 2, 3, 0, 1, 1, w_si=False)
    subiter(2, 0, 1, 2, 3, 2)

    def body(m, carry):
        j0 = 4 * m + 3
        for (dj, a, b, oa, ob, beta) in ((0, 2, 3, 0, 1, 3),
                                         (1, 0, 1, 2, 3, 0),
                                         (2, 2, 3, 0, 1, 1),
                                         (3, 0, 1, 2, 3, 2)):
            subiter(j0 + dj, a, b, oa, ob, beta)
        return carry

    lax.fori_loop(0, (_NPAIR - 8) // 4, body, 0)
    subiter(_NPAIR - 5, 2, 3, 0, 1, 3)
    subiter(_NPAIR - 4, 0, 1, 2, 3, 0)
    subiter(_NPAIR - 3, 2, 3, 0, 1, 1, fetch=False)
    subiter(_NPAIR - 2, 0, 1, 2, 3, 2, fetch=False)
    g_wait(3, 2, 3)
    s_issue(3, 2, 3)
    s_wait(2, 0, 1)
    s_wait(3, 2, 3)
    plsc.subcore_barrier()
    pltpu.sync_copy(acc_sh.at[pl.ds(r0, RT)], acc_hbm.at[pl.ds(h_off, RT)])


_DC_DEPTH = 4


PH = P // 2
PCHH = PH // DEC_CH
PTH = PCHH // NS


@functools.partial(
    pl.kernel,
    out_type=jax.ShapeDtypeStruct((NC * PH, DH), jnp.float32),
    mesh=_MESH,
    scratch_types=(
        [pltpu.VMEM((_DC_DEPTH, 2, DEC_CH), jnp.int32),
         pltpu.VMEM((_DC_DEPTH, DEC_CH, DH), jnp.float32),
         pltpu.VMEM((_DC_DEPTH, DEC_CH, DH), jnp.float32)]
        + [pltpu.SemaphoreType.DMA] * (4 * _DC_DEPTH)
    ),
)
def _dec_kernel(u_hbm, v_hbm, rc_hbm, g_hbm, rc_v, urows_v, vrows_v, *sems):
    sgu = sems[0:_DC_DEPTH]
    sgv = sems[_DC_DEPTH:2 * _DC_DEPTH]
    sw = sems[2 * _DC_DEPTH:3 * _DC_DEPTH]
    si = sems[3 * _DC_DEPTH:4 * _DC_DEPTH]
    c = lax.axis_index("c")
    s = lax.axis_index("s")
    base = c * PCHH + s * PTH
    out0 = c * PH + s * PTH * DEC_CH
    ngrp = PTH // _DC_DEPTH

    def gathers(q):
        pltpu.async_copy(u_hbm.at[rc_v.at[q].at[0]], urows_v.at[q], sgu[q])
        pltpu.async_copy(v_hbm.at[rc_v.at[q].at[1]], vrows_v.at[q], sgv[q])

    def addrows(q):
        def arow(r, cr):
            r0 = 2 * r
            for rr in (r0, r0 + 1):
                for j in range(DH // 16):
                    sl = pl.ds(j * 16, 16)
                    urows_v[q, rr, sl] = urows_v[q, rr, sl] + vrows_v[q, rr, sl]
            return cr
        lax.fori_loop(0, DEC_CH // 2, arow, 0)

    def wout(i, q):
        off = pl.multiple_of(out0 + i * DEC_CH, DEC_CH)
        return g_hbm.at[pl.ds(off, DEC_CH)]

    for q in range(_DC_DEPTH):
        pltpu.sync_copy(rc_hbm.at[base + q], rc_v.at[q])
    for q in range(_DC_DEPTH):
        gathers(q)

    def body(k, carry):
        i0 = pl.multiple_of(k * _DC_DEPTH, _DC_DEPTH)
        nxt = i0 + _DC_DEPTH
        for q in range(_DC_DEPTH):
            pltpu.make_async_copy(u_hbm.at[rc_v.at[q].at[0]], urows_v.at[q],
                                  sgu[q]).wait()
            pltpu.make_async_copy(v_hbm.at[rc_v.at[q].at[1]], vrows_v.at[q],
                                  sgv[q]).wait()
            addrows(q)
            pltpu.async_copy(urows_v.at[q], wout(i0 + q, q), sw[q])
        for q in range(_DC_DEPTH):
            pltpu.make_async_copy(urows_v.at[q], wout(i0 + q, q), sw[q]).wait()
            pltpu.async_copy(rc_hbm.at[base + nxt + q], rc_v.at[q], si[q])
        for q in range(_DC_DEPTH):
            pltpu.make_async_copy(rc_hbm.at[base + nxt + q], rc_v.at[q],
                                  si[q]).wait()
            gathers(q)
        return carry

    lax.fori_loop(0, ngrp - 1, body, 0)
    last = pl.multiple_of((ngrp - 1) * _DC_DEPTH, _DC_DEPTH)
    for q in range(_DC_DEPTH):
        pltpu.make_async_copy(u_hbm.at[rc_v.at[q].at[0]], urows_v.at[q],
                              sgu[q]).wait()
        pltpu.make_async_copy(v_hbm.at[rc_v.at[q].at[1]], vrows_v.at[q],
                              sgv[q]).wait()
        addrows(q)
        pltpu.async_copy(urows_v.at[q], wout(last + q, q), sw[q])
    for q in range(_DC_DEPTH):
        pltpu.make_async_copy(urows_v.at[q], wout(last + q, q), sw[q]).wait()


def _tc1a_body(x_ref, w_ref, h_ref):
    h = jnp.dot(x_ref[...], w_ref[...], preferred_element_type=jnp.float32)
    h_ref[0] = h[:, :DH]
    h_ref[1] = h[:, DH:]


def _tc1a(x_pad, w1):
    return pl.pallas_call(
        _tc1a_body,
        grid=(NP // BR,),
        in_specs=[
            pl.BlockSpec((BR, D), lambda i: (i, 0)),
            pl.BlockSpec((D, D), lambda i: (0, 0)),
        ],
        out_specs=pl.BlockSpec((NC, BR, DH), lambda i: (0, i, 0)),
        out_shape=jax.ShapeDtypeStruct((NC, NP, DH), jnp.float32),
    )(x_pad, w1)


def _tc1b_body(hraw_ref, degs_ref, h_ref, dinv_ref):
    deg = degs_ref[0] + degs_ref[1] + 1.0
    dinv = lax.rsqrt(deg)
    h_ref[0] = hraw_ref[0] * dinv
    h_ref[1] = hraw_ref[1] * dinv
    dinv_ref[...] = dinv


def _tc1b(hraw, degs3):
    return pl.pallas_call(
        _tc1b_body,
        grid=(NP // BR,),
        in_specs=[
            pl.BlockSpec((NC, BR, DH), lambda i: (0, i, 0)),
            pl.BlockSpec((NC, BR, 1), lambda i: (0, i, 0)),
        ],
        out_specs=[
            pl.BlockSpec((NC, BR, DH), lambda i: (0, i, 0)),
            pl.BlockSpec((BR, 1), lambda i: (i, 0)),
        ],
        out_shape=[
            jax.ShapeDtypeStruct((NC, NP, DH), jnp.float32),
            jax.ShapeDtypeStruct((NP, 1), jnp.float32),
        ],
    )(hraw, degs3)


def _tc2_body(acc_ref, dinv_ref, b_ref, w_ref, out_ref):
    accf = jnp.concatenate([acc_ref[0], acc_ref[1]], axis=-1)
    z = jnp.maximum(accf * dinv_ref[...] + b_ref[...], 0.0)
    h = jnp.dot(z, w_ref[...], preferred_element_type=jnp.float32) * dinv_ref[...]
    out_ref[0] = h[:, :DH]
    out_ref[1] = h[:, DH:]


def _tc2(acc3, dinv, b, w):
    return pl.pallas_call(
        _tc2_body,
        grid=(NP // BR,),
        in_specs=[
            pl.BlockSpec((NC, BR, DH), lambda i: (0, i, 0)),
            pl.BlockSpec((BR, 1), lambda i: (i, 0)),
            pl.BlockSpec((1, D), lambda i: (0, 0)),
            pl.BlockSpec((D, D), lambda i: (0, 0)),
        ],
        out_specs=pl.BlockSpec((NC, BR, DH), lambda i: (0, i, 0)),
        out_shape=jax.ShapeDtypeStruct((NC, NP, DH), jnp.float32),
    )(acc3, dinv, b, w)


def _tc3_body(acc_ref, dinv_ref, b_ref, pw1a_ref, pw1b_ref, pb1_ref, u_ref, v_ref):
    accf = jnp.concatenate([acc_ref[0], acc_ref[1]], axis=-1)
    z = jnp.maximum(accf * dinv_ref[...] + b_ref[...], 0.0)
    u = jnp.dot(z, pw1a_ref[...], preferred_element_type=jnp.float32) + pb1_ref[...]
    v = jnp.dot(z, pw1b_ref[...], preferred_element_type=jnp.float32)
    u_ref[0] = u[:, :DH]
    u_ref[1] = u[:, DH:]
    v_ref[0] = v[:, :DH]
    v_ref[1] = v[:, DH:]


def _tc3(acc3, dinv, b, pw1a, pw1b, pb1):
    return pl.pallas_call(
        _tc3_body,
        grid=(NP // BR,),
        in_specs=[
            pl.BlockSpec((NC, BR, DH), lambda i: (0, i, 0)),
            pl.BlockSpec((BR, 1), lambda i: (i, 0)),
            pl.BlockSpec((1, D), lambda i: (0, 0)),
            pl.BlockSpec((D, D), lambda i: (0, 0)),
            pl.BlockSpec((D, D), lambda i: (0, 0)),
            pl.BlockSpec((1, D), lambda i: (0, 0)),
        ],
        out_specs=[
            pl.BlockSpec((NC, BR, DH), lambda i: (0, i, 0)),
            pl.BlockSpec((NC, BR, DH), lambda i: (0, i, 0)),
        ],
        out_shape=[
            jax.ShapeDtypeStruct((NC, NP, DH), jnp.float32),
            jax.ShapeDtypeStruct((NC, NP, DH), jnp.float32),
        ],
    )(acc3, dinv, b, pw1a, pw1b, pb1)


def _tc4_body(g_ref, pw2_ref, pb2_ref, pw3_ref, pb3_ref, out_ref):
    g = jnp.concatenate([g_ref[0], g_ref[1]], axis=-1)
    h1 = jnp.maximum(g, 0.0)
    h2 = jnp.maximum(
        jnp.dot(h1, pw2_ref[...], preferred_element_type=jnp.float32) + pb2_ref[...],
        0.0)
    o = jnp.dot(h2, pw3_ref[...], preferred_element_type=jnp.float32) + pb3_ref[0, 0]
    out_ref[...] = o.reshape(BP // 128, 128)


def _tc4h(g3, pw2, pb2, pw3, pb3):
    return pl.pallas_call(
        _tc4_body,
        grid=(PH // BP,),
        in_specs=[
            pl.BlockSpec((NC, BP, DH), lambda i: (0, i, 0)),
            pl.BlockSpec((D, DH), lambda i: (0, 0)),
            pl.BlockSpec((1, DH), lambda i: (0, 0)),
            pl.BlockSpec((DH, 1), lambda i: (0, 0)),
            pl.BlockSpec((1, 1), lambda i: (0, 0)),
        ],
        out_specs=pl.BlockSpec((BP // 128, 128), lambda i: (i, 0)),
        out_shape=jax.ShapeDtypeStruct((PH // 128, 128), jnp.float32),
    )(g3, pw2, pb2, pw3, pb3)


def kernel(x, edge_index, edge_label_index, W1, b1, W2, b2,
           PW1, PB1, PW2, PB2, PW3, PB3):
    src = edge_index[0].astype(jnp.int32)
    dst = edge_index[1].astype(jnp.int32)
    padidx = N + (jnp.arange(EP - E, dtype=jnp.int32) % (NP - N))
    srcp = jnp.concatenate([src, padidx])
    dstp = jnp.concatenate([dst, padidx])
    d_c = dstp.reshape(ECH, CH)
    s_m = srcp.reshape(MECH, MCH)
    d_m = dstp.reshape(MECH, MCH)
    sd = jnp.concatenate([
        jnp.stack([s_m, d_m], axis=1),
        jnp.stack([s_m + NP, d_m], axis=1),
    ], axis=0)

    row = edge_label_index[0].astype(jnp.int32).reshape(PCH, DEC_CH)
    col = edge_label_index[1].astype(jnp.int32).reshape(PCH, DEC_CH)
    rcs = []
    for h in (0, 1):
        rh = row[h * PCHH:(h + 1) * PCHH]
        ch = col[h * PCHH:(h + 1) * PCHH]
        rcs.append(jnp.concatenate([
            jnp.stack([rh, ch], axis=1),
            jnp.stack([rh + NP, ch + NP], axis=1),
        ], axis=0))

    x_pad = jnp.pad(x, ((0, NP - N), (0, 0)))

    degs = _deg_kernel(d_c)
    hraw = _tc1a(x_pad, W1)
    h1f, dinv = _tc1b(hraw, degs.reshape(NC, NP, 1))
    acc1 = _mp_kernel(h1f.reshape(NC * NP, DH), sd)
    h2f = _tc2(acc1.reshape(NC, NP, DH), dinv, b1.reshape(1, D), W2)
    acc2 = _mp_kernel(h2f.reshape(NC * NP, DH), sd)
    uf, vf = _tc3(acc2.reshape(NC, NP, DH), dinv, b2.reshape(1, D),
                  PW1[:D], PW1[D:], PB1.reshape(1, D))
    uf_f = uf.reshape(NC * NP, DH)
    vf_f = vf.reshape(NC * NP, DH)
    pb2r = PB2.reshape(1, DH)
    pb3r = PB3.reshape(1, 1)
    gA = _dec_kernel(uf_f, vf_f, rcs[0])
    outA = _tc4h(gA.reshape(NC, PH, DH), PW2, pb2r, PW3, pb3r)
    gB = _dec_kernel(uf_f, vf_f, rcs[1])
    outB = _tc4h(gB.reshape(NC, PH, DH), PW2, pb2r, PW3, pb3r)
    return jnp.concatenate([outA, outB], axis=0).reshape(-1)


# speedup vs baseline: 1.0531x; 1.0531x over previous
import functools

import jax
import jax.numpy as jnp
from jax import lax
from jax.experimental import pallas as pl
from jax.experimental.pallas import tpu as pltpu
from jax.experimental.pallas import tpu_sc as plsc

N = 10000
NP = 10240
D = 256
DH = 128
E = 160000
EP = 163840
P = 65536
CH = 128
NC = 2
NS = 16
RT = NP // NS
ECH = EP // CH
DCH = ECH // NC // NS
MCH = 80
MECH = EP // MCH
MET = MECH // NS
DEC_CH = 64
PCH = P // DEC_CH
PT = PCH // NS
BR = 1024
BP = 2048

_MESH = plsc.VectorSubcoreMesh(core_axis_name="c", subcore_axis_name="s",
                               num_cores=NC, num_subcores=NS)


@functools.partial(
    pl.kernel,
    out_type=jax.ShapeDtypeStruct((NC * NP,), jnp.float32),
    mesh=_MESH,
    scratch_types=[
        pltpu.VMEM((DCH, CH), jnp.int32),
        pltpu.VMEM((CH,), jnp.float32),
        pltpu.VMEM((RT,), jnp.float32),
        pltpu.VMEM_SHARED((NP,), jnp.float32),
        pltpu.SemaphoreType.DMA,
    ],
)
def _deg_kernel(dstc_hbm, out_hbm, idx_v, ones_v, zeros_v, hist_sh, sd):
    c = lax.axis_index("c")
    s = lax.axis_index("s")
    for j in range(CH // 16):
        ones_v[pl.ds(j * 16, 16)] = jnp.full((16,), 1.0, jnp.float32)
    for j in range(RT // 16):
        zeros_v[pl.ds(j * 16, 16)] = jnp.zeros((16,), jnp.float32)
    pltpu.sync_copy(zeros_v, hist_sh.at[pl.ds(s * RT, RT)])
    bc = pl.multiple_of((c * NS + s) * DCH, DCH)
    pltpu.sync_copy(dstc_hbm.at[pl.ds(bc, DCH)], idx_v)
    plsc.subcore_barrier()
    for b in range(DCH // 8):
        descs = [
            pltpu.async_copy(ones_v, hist_sh.at[idx_v.at[b * 8 + q]], sd,
                             add=True)
            for q in range(8)
        ]
        for d in descs:
            d.wait()
    plsc.subcore_barrier()
    out_off = pl.multiple_of(c * NP + s * RT, RT)
    pltpu.sync_copy(hist_sh.at[pl.ds(s * RT, RT)], out_hbm.at[pl.ds(out_off, RT)])


_MP_DEPTH = 4
_NPAIR = MET // 2


@functools.partial(
    pl.kernel,
    out_type=jax.ShapeDtypeStruct((NC * NP, DH), jnp.float32),
    mesh=_MESH,
    scratch_types=(
        [pltpu.VMEM((4, 2, 2, MCH), jnp.int32),
         pltpu.VMEM((_MP_DEPTH, MCH, DH), jnp.float32),
         pltpu.VMEM_SHARED((NP, DH), jnp.float32)]
        + [pltpu.SemaphoreType.DMA] * 12
    ),
)
def _mp_kernel(h_hbm, sd_hbm, acc_hbm, sd_v, rows_v, acc_sh, *sems):
    sg = sems[0:4]
    ss = sems[4:8]
    si = sems[8:12]
    c = lax.axis_index("c")
    s = lax.axis_index("s")
    r0 = s * RT
    h_off = pl.multiple_of(c * NP + r0, RT)
    pltpu.sync_copy(h_hbm.at[pl.ds(h_off, RT)], acc_sh.at[pl.ds(r0, RT)])
    plsc.subcore_barrier()
    base = c * MECH + s * MET

    def g_issue(beta, a, b):
        for q, o in ((a, 0), (b, 1)):
            pltpu.async_copy(h_hbm.at[sd_v.at[beta].at[o].at[0]],
                             rows_v.at[q], sg[q])

    def g_wait(beta, a, b):
        for q, o in ((a, 0), (b, 1)):
            pltpu.make_async_copy(h_hbm.at[sd_v.at[beta].at[o].at[0]],
                                  rows_v.at[q], sg[q]).wait()

    def s_issue(beta, a, b):
        for q, o in ((a, 0), (b, 1)):
            pltpu.async_copy(rows_v.at[q],
                             acc_sh.at[sd_v.at[beta].at[o].at[1]], ss[q],
                             add=True)

    def s_wait(beta, a, b):
        for q, o in ((a, 0), (b, 1)):
            pltpu.make_async_copy(rows_v.at[q],
                                  acc_sh.at[sd_v.at[beta].at[o].at[1]],
                                  ss[q]).wait()

    def i_fetch(j, beta):
        jj = pl.multiple_of(2 * j, 2)
        for o in (0, 1):
            pltpu.async_copy(sd_hbm.at[base + jj + o], sd_v.at[beta].at[o],
                             si[beta])

    def i_wait(j, beta):
        jj = pl.multiple_of(2 * j, 2)
        for o in (0, 1):
            pltpu.make_async_copy(sd_hbm.at[base + jj + o],
                                  sd_v.at[beta].at[o], si[beta]).wait()

    def i_sync(j, beta):
        jj = pl.multiple_of(2 * j, 2)
        for o in (0, 1):
            pltpu.sync_copy(sd_hbm.at[base + jj + o], sd_v.at[beta].at[o])

    def subiter(j, a, b, oa, ob, beta, w_ss=True, fetch=True, w_si=True):
        g_wait(beta, a, b)
        s_issue(beta, a, b)
        if w_ss:
            s_wait((beta + 3) % 4, oa, ob)
        if fetch:
            i_fetch(j + 3, (beta + 3) % 4)
        if w_si:
            i_wait(j + 1, (beta + 1) % 4)
        g_issue((beta + 1) % 4, oa, ob)

    i_sync(0, 0)
    i_sync(1, 1)
    i_sync(2, 2)
    g_issue(0, 0, 1)
    subiter(0, 0, 1, 2, 3, 0, w_ss=False, w_si=False)
    subiter(1, 2, 3, 0, 1, 1, w_si=False)
    subiter(2, 0, 1, 2, 3, 2)

    def body(m, carry):
        j0 = 4 * m + 3
        for (dj, a, b, oa, ob, beta) in ((0, 2, 3, 0, 1, 3),
                                         (1, 0, 1, 2, 3, 0),
                                         (2, 2, 3, 0, 1, 1),
                                         (3, 0, 1, 2, 3, 2)):
            subiter(j0 + dj, a, b, oa, ob, beta)
        return carry

    lax.fori_loop(0, (_NPAIR - 8) // 4, body, 0)
    subiter(_NPAIR - 5, 2, 3, 0, 1, 3)
    subiter(_NPAIR - 4, 0, 1, 2, 3, 0)
    subiter(_NPAIR - 3, 2, 3, 0, 1, 1, fetch=False)
    subiter(_NPAIR - 2, 0, 1, 2, 3, 2, fetch=False)
    g_wait(3, 2, 3)
    s_issue(3, 2, 3)
    s_wait(2, 0, 1)
    s_wait(3, 2, 3)
    plsc.subcore_barrier()
    pltpu.sync_copy(acc_sh.at[pl.ds(r0, RT)], acc_hbm.at[pl.ds(h_off, RT)])


_DC_DEPTH = 4
_DNPAIR = PT // 2


@functools.partial(
    pl.kernel,
    out_type=jax.ShapeDtypeStruct((NC * P, DH), jnp.float32),
    mesh=_MESH,
    scratch_types=(
        [pltpu.VMEM((4, 2, 2, DEC_CH), jnp.int32),
         pltpu.VMEM((_DC_DEPTH, DEC_CH, DH), jnp.float32),
         pltpu.VMEM((_DC_DEPTH, DEC_CH, DH), jnp.float32)]
        + [pltpu.SemaphoreType.DMA] * 16
    ),
)
def _dec_kernel(u_hbm, v_hbm, rc_hbm, g_hbm, rc_v, urows_v, vrows_v, *sems):
    sgu = sems[0:4]
    sgv = sems[4:8]
    sw = sems[8:12]
    si = sems[12:16]
    c = lax.axis_index("c")
    s = lax.axis_index("s")
    base = c * PCH + s * PT
    out0 = c * P + s * PT * DEC_CH

    def g_issue(beta, a, b):
        for q, o in ((a, 0), (b, 1)):
            pltpu.async_copy(u_hbm.at[rc_v.at[beta].at[o].at[0]],
                             urows_v.at[q], sgu[q])
            pltpu.async_copy(v_hbm.at[rc_v.at[beta].at[o].at[1]],
                             vrows_v.at[q], sgv[q])

    def g_wait(beta, a, b):
        for q, o in ((a, 0), (b, 1)):
            pltpu.make_async_copy(u_hbm.at[rc_v.at[beta].at[o].at[0]],
                                  urows_v.at[q], sgu[q]).wait()
            pltpu.make_async_copy(v_hbm.at[rc_v.at[beta].at[o].at[1]],
                                  vrows_v.at[q], sgv[q]).wait()

    def addrows(q):
        def arow(r, cr):
            rr0 = 2 * r
            for rr in (rr0, rr0 + 1):
                for j in range(DH // 16):
                    sl = pl.ds(j * 16, 16)
                    urows_v[q, rr, sl] = urows_v[q, rr, sl] + vrows_v[q, rr, sl]
            return cr
        lax.fori_loop(0, DEC_CH // 2, arow, 0)

    def wdst(i):
        off = pl.multiple_of(out0 + i * DEC_CH, DEC_CH)
        return g_hbm.at[pl.ds(off, DEC_CH)]

    def w_issue(j, a, b):
        jj = pl.multiple_of(2 * j, 2)
        for q, o in ((a, 0), (b, 1)):
            pltpu.async_copy(urows_v.at[q], wdst(jj + o), sw[q])

    def w_wait(j, a, b):
        jj = pl.multiple_of(2 * j, 2)
        for q, o in ((a, 0), (b, 1)):
            pltpu.make_async_copy(urows_v.at[q], wdst(jj + o), sw[q]).wait()

    def i_fetch(j, beta):
        jj = pl.multiple_of(2 * j, 2)
        for o in (0, 1):
            pltpu.async_copy(rc_hbm.at[base + jj + o], rc_v.at[beta].at[o],
                             si[beta])

    def i_wait(j, beta):
        jj = pl.multiple_of(2 * j, 2)
        for o in (0, 1):
            pltpu.make_async_copy(rc_hbm.at[base + jj + o],
                                  rc_v.at[beta].at[o], si[beta]).wait()

    def i_sync(j, beta):
        jj = pl.multiple_of(2 * j, 2)
        for o in (0, 1):
            pltpu.sync_copy(rc_hbm.at[base + jj + o], rc_v.at[beta].at[o])

    def subiter(j, a, b, oa, ob, beta,
                w_w=True, w_si=True, fetch=True, gather=True):
        g_wait(beta, a, b)
        if w_w:
            w_wait(j - 1, oa, ob)
        if gather:
            if w_si:
                i_wait(j + 1, (beta + 1) % 4)
            g_issue((beta + 1) % 4, oa, ob)
        addrows(a)
        addrows(b)
        w_issue(j, a, b)
        if fetch:
            i_fetch(j + 3, (beta + 3) % 4)

    i_sync(0, 0)
    i_sync(1, 1)
    i_sync(2, 2)
    g_issue(0, 0, 1)
    subiter(0, 0, 1, 2, 3, 0, w_w=False, w_si=False)
    subiter(1, 2, 3, 0, 1, 1, w_si=False)
    subiter(2, 0, 1, 2, 3, 2)

    def body(m, carry):
        j0 = 4 * m + 3
        for (dj, a, b, oa, ob, beta) in ((0, 2, 3, 0, 1, 3),
                                         (1, 0, 1, 2, 3, 0),
                                         (2, 2, 3, 0, 1, 1),
                                         (3, 0, 1, 2, 3, 2)):
            subiter(j0 + dj, a, b, oa, ob, beta)
        return carry

    lax.fori_loop(0, (_DNPAIR - 8) // 4, body, 0)
    subiter(_DNPAIR - 5, 2, 3, 0, 1, 3)
    subiter(_DNPAIR - 4, 0, 1, 2, 3, 0)
    subiter(_DNPAIR - 3, 2, 3, 0, 1, 1, fetch=False)
    subiter(_DNPAIR - 2, 0, 1, 2, 3, 2, fetch=False)
    subiter(_DNPAIR - 1, 2, 3, 0, 1, 3, fetch=False, gather=False)
    w_wait(_DNPAIR - 1, 2, 3)


def _tc1_body(x_ref, w_ref, degs_ref, h_ref, dinv_ref):
    deg = degs_ref[0] + degs_ref[1] + 1.0
    dinv = lax.rsqrt(deg)
    h = jnp.dot(x_ref[...], w_ref[...], preferred_element_type=jnp.float32) * dinv
    h_ref[0] = h[:, :DH]
    h_ref[1] = h[:, DH:]
    dinv_ref[...] = dinv


def _tc1(x_pad, w1, degs3):
    return pl.pallas_call(
        _tc1_body,
        grid=(NP // BR,),
        in_specs=[
            pl.BlockSpec((BR, D), lambda i: (i, 0)),
            pl.BlockSpec((D, D), lambda i: (0, 0)),
            pl.BlockSpec((NC, BR, 1), lambda i: (0, i, 0)),
        ],
        out_specs=[
            pl.BlockSpec((NC, BR, DH), lambda i: (0, i, 0)),
            pl.BlockSpec((BR, 1), lambda i: (i, 0)),
        ],
        out_shape=[
            jax.ShapeDtypeStruct((NC, NP, DH), jnp.float32),
            jax.ShapeDtypeStruct((NP, 1), jnp.float32),
        ],
    )(x_pad, w1, degs3)


def _tc2_body(acc_ref, dinv_ref, b_ref, w_ref, out_ref):
    accf = jnp.concatenate([acc_ref[0], acc_ref[1]], axis=-1)
    z = jnp.maximum(accf * dinv_ref[...] + b_ref[...], 0.0)
    h = jnp.dot(z, w_ref[...], preferred_element_type=jnp.float32) * dinv_ref[...]
    out_ref[0] = h[:, :DH]
    out_ref[1] = h[:, DH:]


def _tc2(acc3, dinv, b, w):
    return pl.pallas_call(
        _tc2_body,
        grid=(NP // BR,),
        in_specs=[
            pl.BlockSpec((NC, BR, DH), lambda i: (0, i, 0)),
            pl.BlockSpec((BR, 1), lambda i: (i, 0)),
            pl.BlockSpec((1, D), lambda i: (0, 0)),
            pl.BlockSpec((D, D), lambda i: (0, 0)),
        ],
        out_specs=pl.BlockSpec((NC, BR, DH), lambda i: (0, i, 0)),
        out_shape=jax.ShapeDtypeStruct((NC, NP, DH), jnp.float32),
    )(acc3, dinv, b, w)


def _tc3_body(acc_ref, dinv_ref, b_ref, pw1a_ref, pw1b_ref, pb1_ref, u_ref, v_ref):
    accf = jnp.concatenate([acc_ref[0], acc_ref[1]], axis=-1)
    z = jnp.maximum(accf * dinv_ref[...] + b_ref[...], 0.0)
    u = jnp.dot(z, pw1a_ref[...], preferred_element_type=jnp.float32) + pb1_ref[...]
    v = jnp.dot(z, pw1b_ref[...], preferred_element_type=jnp.float32)
    u_ref[0] = u[:, :DH]
    u_ref[1] = u[:, DH:]
    v_ref[0] = v[:, :DH]
    v_ref[1] = v[:, DH:]


def _tc3(acc3, dinv, b, pw1a, pw1b, pb1):
    return pl.pallas_call(
        _tc3_body,
        grid=(NP // BR,),
        in_specs=[
            pl.BlockSpec((NC, BR, DH), lambda i: (0, i, 0)),
            pl.BlockSpec((BR, 1), lambda i: (i, 0)),
            pl.BlockSpec((1, D), lambda i: (0, 0)),
            pl.BlockSpec((D, D), lambda i: (0, 0)),
            pl.BlockSpec((D, D), lambda i: (0, 0)),
            pl.BlockSpec((1, D), lambda i: (0, 0)),
        ],
        out_specs=[
            pl.BlockSpec((NC, BR, DH), lambda i: (0, i, 0)),
            pl.BlockSpec((NC, BR, DH), lambda i: (0, i, 0)),
        ],
        out_shape=[
            jax.ShapeDtypeStruct((NC, NP, DH), jnp.float32),
            jax.ShapeDtypeStruct((NC, NP, DH), jnp.float32),
        ],
    )(acc3, dinv, b, pw1a, pw1b, pb1)


def _tc4_body(g_ref, pw2_ref, pb2_ref, pw3_ref, pb3_ref, out_ref):
    g = jnp.concatenate([g_ref[0], g_ref[1]], axis=-1)
    h1 = jnp.maximum(g, 0.0)
    h2 = jnp.maximum(
        jnp.dot(h1, pw2_ref[...], preferred_element_type=jnp.float32) + pb2_ref[...],
        0.0)
    o = jnp.dot(h2, pw3_ref[...], preferred_element_type=jnp.float32) + pb3_ref[0, 0]
    out_ref[...] = o.reshape(BP // 128, 128)


def _tc4(g3, pw2, pb2, pw3, pb3):
    return pl.pallas_call(
        _tc4_body,
        grid=(P // BP,),
        in_specs=[
            pl.BlockSpec((NC, BP, DH), lambda i: (0, i, 0)),
            pl.BlockSpec((D, DH), lambda i: (0, 0)),
            pl.BlockSpec((1, DH), lambda i: (0, 0)),
            pl.BlockSpec((DH, 1), lambda i: (0, 0)),
            pl.BlockSpec((1, 1), lambda i: (0, 0)),
        ],
        out_specs=pl.BlockSpec((BP // 128, 128), lambda i: (i, 0)),
        out_shape=jax.ShapeDtypeStruct((P // 128, 128), jnp.float32),
    )(g3, pw2, pb2, pw3, pb3)


def kernel(x, edge_index, edge_label_index, W1, b1, W2, b2,
           PW1, PB1, PW2, PB2, PW3, PB3):
    src = edge_index[0].astype(jnp.int32)
    dst = edge_index[1].astype(jnp.int32)
    padidx = N + (jnp.arange(EP - E, dtype=jnp.int32) % (NP - N))
    srcp = jnp.concatenate([src, padidx])
    dstp = jnp.concatenate([dst, padidx])
    d_c = dstp.reshape(ECH, CH)
    s_m = srcp.reshape(MECH, MCH)
    d_m = dstp.reshape(MECH, MCH)
    sd = jnp.concatenate([
        jnp.stack([s_m, d_m], axis=1),
        jnp.stack([s_m + NP, d_m], axis=1),
    ], axis=0)

    row = edge_label_index[0].astype(jnp.int32).reshape(PCH, DEC_CH)
    col = edge_label_index[1].astype(jnp.int32).reshape(PCH, DEC_CH)
    rc = jnp.concatenate([
        jnp.stack([row, col], axis=1),
        jnp.stack([row + NP, col + NP], axis=1),
    ], axis=0)

    x_pad = jnp.pad(x, ((0, NP - N), (0, 0)))

    degs = _deg_kernel(d_c)
    h1f, dinv = _tc1(x_pad, W1, degs.reshape(NC, NP, 1))
    acc1 = _mp_kernel(h1f.reshape(NC * NP, DH), sd)
    h2f = _tc2(acc1.reshape(NC, NP, DH), dinv, b1.reshape(1, D), W2)
    acc2 = _mp_kernel(h2f.reshape(NC * NP, DH), sd)
    uf, vf = _tc3(acc2.reshape(NC, NP, DH), dinv, b2.reshape(1, D),
                  PW1[:D], PW1[D:], PB1.reshape(1, D))
    g = _dec_kernel(uf.reshape(NC * NP, DH), vf.reshape(NC * NP, DH), rc)
    out = _tc4(g.reshape(NC, P, DH), PW2, PB2.reshape(1, DH),
               PW3, PB3.reshape(1, 1))
    return out.reshape(-1)
